# Initial kernel scaffold; baseline (speedup 1.0000x reference)
import jax
import jax.numpy as jnp
from jax.experimental import pallas as pl


def kernel(u_node, u_edge, node_base_log_probs, edge_base_log_probs):
    raise NotImplementedError("write your pallas kernel here")


# speedup vs baseline: 6.6996x; 6.6996x over previous
import jax
import jax.numpy as jnp
from jax import lax
from jax.experimental import pallas as pl
from jax.experimental.pallas import tpu as pltpu
from jax.experimental.pallas import tpu_sc as plsc

MAX_SIZE = 38
NODE_DIM = 9
BOND_DIM = 4
N_EDGES = 378
B = 16384
NODE_W = MAX_SIZE * NODE_DIM
EDGE_W = N_EDGES * BOND_DIM
NODE_WP = 352

NW = 32
ROWS_W = B // NW
GROUPS = ROWS_W // 16

_LN2 = 0.6931471805599453
_SQRTH = 0.7071067811865476
_C3 = 2.0 / 3.0
_C5 = 2.0 / 5.0
_C7 = 2.0 / 7.0


def _log(u):
    bits = lax.bitcast_convert_type(u, jnp.int32)
    e = (bits >> 23) - 126
    m = lax.bitcast_convert_type(
        (bits & 0x007FFFFF) | 0x3F000000, jnp.float32)
    cond = m < _SQRTH
    m = jnp.where(cond, m + m, m)
    ef = (e - cond.astype(jnp.int32)).astype(jnp.float32)
    r = (m - 1.0) / (m + 1.0)
    r2 = r * r
    w = ((_C7 * r2 + _C5) * r2 + _C3) * r2 + 2.0
    return ef * _LN2 + r * w


def _argmin_step(j, s, best, bj):
    lt = s < best
    return jnp.where(lt, s, best), jnp.where(lt, jnp.int32(j), bj)


def _body(un, ue, nnegc, nlp, enegc, elp,
          out_lp, out_noh, out_eoh,
          ubn0, ubn1, ube0, ube1, ohn0, ohn1, ohe0, ohe1,
          tnc, tnl, tec, tel, acc0, acc1, sems):
    wid = lax.axis_index("s") * 2 + lax.axis_index("c")

    pltpu.sync_copy(nnegc, tnc)
    pltpu.sync_copy(nlp, tnl)
    pltpu.sync_copy(enegc, tec)
    pltpu.sync_copy(elp, tel)

    lane = lax.iota(jnp.int32, 16)
    base_n = lane * NODE_W
    base_e = lane * EDGE_W

    slots = ((ubn0, ube0, ohn0, ohe0, acc0, 0),
             (ubn1, ube1, ohn1, ohe1, acc1, 1))

    def in_copies(g, slot):
        ubn, ube = slots[slot][0], slots[slot][1]
        r0 = wid * ROWS_W + g * 16
        return (pltpu.make_async_copy(un.at[pl.ds(r0 * NODE_W, 16 * NODE_W)],
                                      ubn, sems.at[slot, 0]),
                pltpu.make_async_copy(ue.at[pl.ds(r0 * EDGE_W, 16 * EDGE_W)],
                                      ube, sems.at[slot, 1]))

    def out_copies(g, slot):
        _, _, ohn, ohe, acc, _ = slots[slot]
        r0 = wid * ROWS_W + g * 16
        return (pltpu.make_async_copy(ohn, out_noh.at[pl.ds(r0 * NODE_W, 16 * NODE_W)],
                                      sems.at[slot, 2]),
                pltpu.make_async_copy(ohe, out_eoh.at[pl.ds(r0 * EDGE_W, 16 * EDGE_W)],
                                      sems.at[slot, 3]),
                pltpu.make_async_copy(acc, out_lp.at[pl.ds(r0, 16)],
                                      sems.at[slot, 4]))

    def start_in(g, slot):
        for c in in_copies(g, slot):
            c.start()

    def wait_in(g, slot):
        for c in in_copies(g, slot):
            c.wait()

    def start_out(g, slot):
        for c in out_copies(g, slot):
            c.start()

    def wait_out(g, slot):
        for c in out_copies(g, slot):
            c.wait()

    def compute(g, slot):
        ubn, ube, ohn, ohe, accb, _ = slots[slot]

        def node_pos(i, acc):
            off = i * NODE_DIM
            best = jnp.full((16,), jnp.float32(jnp.inf))
            bj = jnp.zeros((16,), jnp.int32)
            for j in range(NODE_DIM):
                u = plsc.load_gather(ubn, [base_n + (off + j)])
                u = jnp.maximum(u, 1e-10)
                negc = plsc.load_gather(tnc, [jnp.full((16,), off + j, jnp.int32)])
                s = _log(u) * negc
                best, bj = _argmin_step(j, s, best, bj)
            ll = plsc.load_gather(tnl, [off + bj])
            for j in range(NODE_DIM):
                plsc.store_scatter(ohn, [base_n + (off + j)],
                                   (bj == j).astype(jnp.float32))
            return acc + ll

        acc = lax.fori_loop(0, MAX_SIZE, node_pos, jnp.zeros((16,), jnp.float32))

        def edge_pos(e, acc):
            off = e * BOND_DIM
            best = jnp.full((16,), jnp.float32(jnp.inf))
            bj = jnp.zeros((16,), jnp.int32)
            for j in range(BOND_DIM):
                u = plsc.load_gather(ube, [base_e + (off + j)])
                u = jnp.maximum(u, 1e-10)
                negc = plsc.load_gather(tec, [jnp.full((16,), off + j, jnp.int32)])
                s = _log(u) * negc
                best, bj = _argmin_step(j, s, best, bj)
            ll = plsc.load_gather(tel, [off + bj])
            for j in range(BOND_DIM):
                plsc.store_scatter(ohe, [base_e + (off + j)],
                                   (bj == j).astype(jnp.float32))
            return acc + ll

        acc = lax.fori_loop(0, N_EDGES, edge_pos, acc)
        accb[...] = acc

    start_in(0, 0)

    def pair(p, _):
        for k in range(2):
            g = p * 2 + k

            @pl.when(g + 1 < GROUPS)
            def _():
                start_in(g + 1, 1 - k)

            wait_in(g, k)

            @pl.when(g >= 2)
            def _():
                wait_out(g - 2, k)

            compute(g, k)
            start_out(g, k)
        return 0

    lax.fori_loop(0, GROUPS // 2, pair, 0)
    wait_out(GROUPS - 2, 0)
    wait_out(GROUPS - 1, 1)


@jax.jit
def kernel(u_node, u_edge, node_base_log_probs, edge_base_log_probs):
    nl = node_base_log_probs * 0.3
    el = edge_base_log_probs / 0.3
    n_negc = jnp.pad(-jnp.exp(-nl).reshape(-1), (0, NODE_WP - NODE_W),
                     constant_values=-1.0)
    n_lp = jnp.pad(jax.nn.log_softmax(nl, axis=-1).reshape(-1),
                   (0, NODE_WP - NODE_W))
    e_negc = -jnp.exp(-el).reshape(-1)
    e_lp = jax.nn.log_softmax(el, axis=-1).reshape(-1)

    mesh = plsc.VectorSubcoreMesh(core_axis_name="c", subcore_axis_name="s")
    call = pl.kernel(
        _body,
        out_type=[
            jax.ShapeDtypeStruct((B,), jnp.float32),
            jax.ShapeDtypeStruct((B * NODE_W,), jnp.float32),
            jax.ShapeDtypeStruct((B * EDGE_W,), jnp.float32),
        ],
        mesh=mesh,
        compiler_params=pltpu.CompilerParams(needs_layout_passes=False),
        scratch_types=[
            pltpu.VMEM((16 * NODE_W,), jnp.float32),
            pltpu.VMEM((16 * NODE_W,), jnp.float32),
            pltpu.VMEM((16 * EDGE_W,), jnp.float32),
            pltpu.VMEM((16 * EDGE_W,), jnp.float32),
            pltpu.VMEM((16 * NODE_W,), jnp.float32),
            pltpu.VMEM((16 * NODE_W,), jnp.float32),
            pltpu.VMEM((16 * EDGE_W,), jnp.float32),
            pltpu.VMEM((16 * EDGE_W,), jnp.float32),
            pltpu.VMEM((NODE_WP,), jnp.float32),
            pltpu.VMEM((NODE_WP,), jnp.float32),
            pltpu.VMEM((EDGE_W,), jnp.float32),
            pltpu.VMEM((EDGE_W,), jnp.float32),
            pltpu.VMEM((16,), jnp.float32),
            pltpu.VMEM((16,), jnp.float32),
            pltpu.SemaphoreType.DMA((2, 5)),
        ],
    )
    tlp, noh, eoh = call(u_node.reshape(-1), u_edge.reshape(-1),
                         n_negc, n_lp, e_negc, e_lp)
    return (tlp,
            noh.reshape(B, MAX_SIZE, NODE_DIM),
            eoh.reshape(B, N_EDGES, BOND_DIM))
